# baseline (speedup 1.0000x reference)
import functools

import jax
import jax.numpy as jnp
from jax import lax
from jax.experimental import pallas as pl
from jax.experimental.pallas import tpu as pltpu
from jax.experimental.pallas import tpu_sc as plsc

D_MODEL = 1024
NUM_CORES = 2
NUM_SUBCORES = 16
NW = NUM_CORES * NUM_SUBCORES
CHUNK = 8
NBUF = 8


def _build_gather(batch):
    bpw = batch // NW
    nch = bpw // CHUNK
    mesh = plsc.VectorSubcoreMesh(core_axis_name="c", subcore_axis_name="s")

    @functools.partial(
        pl.kernel,
        out_type=jax.ShapeDtypeStruct((batch, D_MODEL), jnp.float32),
        mesh=mesh,
        scratch_types=[
            pltpu.VMEM((nch, CHUNK), jnp.int32),
            pltpu.VMEM((NBUF, CHUNK, D_MODEL), jnp.float32),
            [pltpu.SemaphoreType.DMA] * NBUF,
            [pltpu.SemaphoreType.DMA] * NBUF,
        ],
    )
    def gather_kernel(idx_hbm, table_hbm, out_hbm, idx_v, buf, gsems, ssems):
        wid = lax.axis_index("s") * NUM_CORES + lax.axis_index("c")
        base = wid * bpw

        def start_gather(c, b):
            pltpu.async_copy(table_hbm.at[idx_v.at[c]], buf.at[b], gsems[b])

        def wait_gather(c, b):
            pltpu.make_async_copy(
                table_hbm.at[idx_v.at[c]], buf.at[b], gsems[b]
            ).wait()

        def start_scatter(c, b):
            pltpu.async_copy(
                buf.at[b], out_hbm.at[pl.ds(base + c * CHUNK, CHUNK)], ssems[b]
            )

        def wait_scatter(c, b):
            pltpu.make_async_copy(
                buf.at[b], out_hbm.at[pl.ds(base + c * CHUNK, CHUNK)], ssems[b]
            ).wait()

        pltpu.sync_copy(idx_hbm.at[wid], idx_v)
        for b in range(NBUF // 2):
            start_gather(b, b)

        @pl.loop(0, nch, step=NBUF)
        def _(g):
            for b in range(NBUF):
                c = g + b
                wait_gather(c, b)
                start_scatter(c, b)
                c2 = c + NBUF // 2
                b2 = (b + NBUF // 2) % NBUF

                @pl.when(jnp.logical_and(c2 - NBUF >= 0, c2 < nch))
                def _():
                    wait_scatter(c2 - NBUF, b2)

                @pl.when(c2 < nch)
                def _():
                    start_gather(c2, b2)

        for b in range(NBUF):
            wait_scatter(nch - NBUF + b, (nch - NBUF + b) % NBUF)

    return gather_kernel


def kernel(position_ids, pe_weight):
    bsz, seq = position_ids.shape
    batch = bsz * seq
    idx = position_ids.reshape(NW, batch // NW // CHUNK, CHUNK).astype(jnp.int32)
    out = _build_gather(batch)(idx, pe_weight)
    return out.reshape(bsz, seq, D_MODEL)


# speedup vs baseline: 1.0099x; 1.0099x over previous
import functools

import jax
import jax.numpy as jnp
from jax import lax
from jax.experimental import pallas as pl
from jax.experimental.pallas import tpu as pltpu
from jax.experimental.pallas import tpu_sc as plsc

D_MODEL = 1024
NUM_CORES = 2
NUM_SUBCORES = 16
NW = NUM_CORES * NUM_SUBCORES
CHUNK = 16
NBUF = 4
LA = NBUF // 2


def _build_gather(bsz, seq):
    batch = bsz * seq
    bpw = batch // NW
    nch = bpw // CHUNK
    wpr = seq // bpw
    mesh = plsc.VectorSubcoreMesh(core_axis_name="c", subcore_axis_name="s")

    @functools.partial(
        pl.kernel,
        out_type=jax.ShapeDtypeStruct((batch, D_MODEL), jnp.float32),
        mesh=mesh,
        scratch_types=[
            pltpu.VMEM((bpw,), jnp.int32),
            pltpu.VMEM((NBUF, CHUNK, D_MODEL), jnp.float32),
            [pltpu.SemaphoreType.DMA] * NBUF,
            [pltpu.SemaphoreType.DMA] * NBUF,
        ],
    )
    def gather_kernel(idx_hbm, table_hbm, out_hbm, idx_v, buf, gsems, ssems):
        wid = lax.axis_index("s") * NUM_CORES + lax.axis_index("c")
        base = wid * bpw

        def start_gather(c, b):
            pltpu.async_copy(
                table_hbm.at[idx_v.at[pl.ds(c * CHUNK, CHUNK)]], buf.at[b], gsems[b]
            )

        def wait_gather(c, b):
            pltpu.make_async_copy(
                table_hbm.at[idx_v.at[pl.ds(c * CHUNK, CHUNK)]], buf.at[b], gsems[b]
            ).wait()

        def start_scatter(c, b):
            pltpu.async_copy(
                buf.at[b], out_hbm.at[pl.ds(base + c * CHUNK, CHUNK)], ssems[b]
            )

        def wait_scatter(c, b):
            pltpu.make_async_copy(
                buf.at[b], out_hbm.at[pl.ds(base + c * CHUNK, CHUNK)], ssems[b]
            ).wait()

        pltpu.sync_copy(
            idx_hbm.at[wid // wpr].at[pl.ds((wid % wpr) * bpw, bpw)], idx_v
        )
        for b in range(LA):
            start_gather(b, b)

        @pl.loop(0, nch, step=NBUF)
        def _(g):
            for b in range(NBUF):
                c = g + b
                wait_gather(c, b)
                start_scatter(c, b)
                c2 = c + LA
                b2 = (b + LA) % NBUF

                @pl.when(jnp.logical_and(c2 - NBUF >= 0, c2 < nch))
                def _():
                    wait_scatter(c2 - NBUF, b2)

                @pl.when(c2 < nch)
                def _():
                    start_gather(c2, b2)

        for b in range(NBUF):
            wait_scatter(nch - NBUF + b, (nch - NBUF + b) % NBUF)

    return gather_kernel


def kernel(position_ids, pe_weight):
    bsz, seq = position_ids.shape
    out = _build_gather(bsz, seq)(position_ids.astype(jnp.int32), pe_weight)
    return out.reshape(bsz, seq, D_MODEL)


# speedup vs baseline: 1.0108x; 1.0009x over previous
import functools

import jax
import jax.numpy as jnp
from jax import lax
from jax.experimental import pallas as pl
from jax.experimental.pallas import tpu as pltpu
from jax.experimental.pallas import tpu_sc as plsc

D_MODEL = 1024
NUM_CORES = 2
NUM_SUBCORES = 16
NW = NUM_CORES * NUM_SUBCORES
CHUNK = 16
NBUF = 4
LA = NBUF // 2


def _build_gather(bsz, seq):
    batch = bsz * seq
    bpw = batch // NW
    nch = bpw // CHUNK
    wpr = seq // bpw
    mesh = plsc.VectorSubcoreMesh(core_axis_name="c", subcore_axis_name="s")

    @functools.partial(
        pl.kernel,
        out_type=jax.ShapeDtypeStruct((batch, D_MODEL), jnp.float32),
        mesh=mesh,
        scratch_types=[
            pltpu.VMEM((bpw,), jnp.int32),
            pltpu.VMEM((NBUF, CHUNK, D_MODEL), jnp.float32),
            [pltpu.SemaphoreType.DMA] * NBUF,
            [pltpu.SemaphoreType.DMA] * NBUF,
        ],
    )
    def gather_kernel(idx_hbm, table_hbm, out_hbm, idx_v, buf, gsems, ssems):
        wid = lax.axis_index("s") * NUM_CORES + lax.axis_index("c")
        base = wid * bpw

        def start_gather(c, b):
            pltpu.async_copy(
                table_hbm.at[idx_v.at[pl.ds(c * CHUNK, CHUNK)]], buf.at[b], gsems[b],
                priority=1,
            )

        def wait_gather(c, b):
            pltpu.make_async_copy(
                table_hbm.at[idx_v.at[pl.ds(c * CHUNK, CHUNK)]], buf.at[b], gsems[b]
            ).wait()

        def start_scatter(c, b):
            pltpu.async_copy(
                buf.at[b], out_hbm.at[pl.ds(base + c * CHUNK, CHUNK)], ssems[b]
            )

        def wait_scatter(c, b):
            pltpu.make_async_copy(
                buf.at[b], out_hbm.at[pl.ds(base + c * CHUNK, CHUNK)], ssems[b]
            ).wait()

        pltpu.sync_copy(
            idx_hbm.at[wid // wpr].at[pl.ds((wid % wpr) * bpw, bpw)], idx_v
        )
        for b in range(LA):
            start_gather(b, b)

        @pl.loop(0, nch, step=NBUF)
        def _(g):
            for b in range(NBUF):
                c = g + b
                wait_gather(c, b)
                start_scatter(c, b)
                c2 = c + LA
                b2 = (b + LA) % NBUF

                @pl.when(jnp.logical_and(c2 - NBUF >= 0, c2 < nch))
                def _():
                    wait_scatter(c2 - NBUF, b2)

                @pl.when(c2 < nch)
                def _():
                    start_gather(c2, b2)

        for b in range(NBUF):
            wait_scatter(nch - NBUF + b, (nch - NBUF + b) % NBUF)

    return gather_kernel


def kernel(position_ids, pe_weight):
    bsz, seq = position_ids.shape
    out = _build_gather(bsz, seq)(position_ids.astype(jnp.int32), pe_weight)
    return out.reshape(bsz, seq, D_MODEL)
